# baseline (speedup 1.0000x reference)
import functools

import jax
import jax.numpy as jnp
from jax import lax
from jax.experimental import pallas as pl
from jax.experimental.pallas import tpu as pltpu
from jax.experimental.pallas import tpu_sc as plsc

NC = 2
NS = 16
NW = NC * NS
LANES = 16
CH = 128
NBUF = 2


def _tanh16(v):
    c = jnp.clip(v, -10.0, 10.0)
    e = jnp.exp(-2.0 * c)
    return (1.0 - e) / (1.0 + e)


@functools.lru_cache(maxsize=None)
def _build(N, D, n_per_w, n_ch):
    mesh = plsc.VectorSubcoreMesh(core_axis_name="c", subcore_axis_name="s")

    @functools.partial(
        pl.kernel,
        out_type=jax.ShapeDtypeStruct((N, D), jnp.float32),
        mesh=mesh,
        scratch_types=[
            pltpu.VMEM((n_ch, CH), jnp.int32),
            pltpu.VMEM((NBUF, CH, D), jnp.float32),
            pltpu.SemaphoreType.DMA((NBUF,)),
        ],
        compiler_params=pltpu.CompilerParams(use_tc_tiling_on_sc=False),
    )
    def k(x_hbm, table_hbm, out_hbm, idx_v, buf_v, sems):
        wid = lax.axis_index("s") * NC + lax.axis_index("c")
        base = wid * n_per_w
        pltpu.sync_copy(x_hbm.at[wid], idx_v)
        for b in range(NBUF):
            pltpu.async_copy(table_hbm.at[idx_v.at[b]], buf_v.at[b], sems.at[b])

        def chunk_body(ch, carry):
            for b in range(NBUF):
                pltpu.make_async_copy(
                    table_hbm.at[idx_v.at[b]], buf_v.at[b], sems.at[b]
                ).wait()

                def row_body(i, c):
                    for h in range(D // LANES):
                        v = buf_v[b, i, pl.ds(h * LANES, LANES)]
                        buf_v[b, i, pl.ds(h * LANES, LANES)] = _tanh16(v)
                    return c

                lax.fori_loop(0, CH, row_body, 0, unroll=4)
                pltpu.sync_copy(
                    buf_v.at[b], out_hbm.at[pl.ds(base + (ch + b) * CH, CH)]
                )
                nxt = ch + b + NBUF

                @pl.when(nxt < n_ch)
                def _():
                    pltpu.async_copy(
                        table_hbm.at[idx_v.at[nxt]], buf_v.at[b], sems.at[b]
                    )

            return carry

        lax.fori_loop(0, n_ch // NBUF, lambda g, c: chunk_body(g * NBUF, c), 0)

    return k


def kernel(x, table):
    B, S = x.shape
    V, D = table.shape
    N = B * S
    n_per_w = N // NW
    n_ch = n_per_w // CH
    xw = x.reshape(NW, n_ch, CH).astype(jnp.int32)
    out = _build(N, D, n_per_w, n_ch)(xw, table)
    return out.reshape(B, S, D)


# speedup vs baseline: 1.0109x; 1.0109x over previous
import functools

import jax
import jax.numpy as jnp
from jax import lax
from jax.experimental import pallas as pl
from jax.experimental.pallas import tpu as pltpu
from jax.experimental.pallas import tpu_sc as plsc

NC = 2
NS = 16
NW = NC * NS
LANES = 16
CH = 128
NBUF = 2


def _tanh16(v):
    c = jnp.clip(v, -10.0, 10.0)
    e = jnp.exp(-2.0 * c)
    return (1.0 - e) / (1.0 + e)


@functools.lru_cache(maxsize=None)
def _build(N, D, n_per_w, n_ch):
    mesh = plsc.VectorSubcoreMesh(core_axis_name="c", subcore_axis_name="s")

    @functools.partial(
        pl.kernel,
        out_type=jax.ShapeDtypeStruct((N, D), jnp.float32),
        mesh=mesh,
        scratch_types=[
            pltpu.VMEM((n_ch, CH), jnp.int32),
            pltpu.VMEM((NBUF, CH, D), jnp.float32),
            pltpu.SemaphoreType.DMA((NBUF,)),
        ],
        compiler_params=pltpu.CompilerParams(use_tc_tiling_on_sc=False),
    )
    def k(x_hbm, table_hbm, out_hbm, idx_v, buf_v, sems):
        wid = lax.axis_index("s") * NC + lax.axis_index("c")
        base = wid * n_per_w
        pltpu.sync_copy(x_hbm.at[wid], idx_v)
        for b in range(NBUF):
            pltpu.async_copy(table_hbm.at[idx_v.at[b]], buf_v.at[b], sems.at[b])

        def chunk_body(ch, carry):
            for b in range(NBUF):
                pltpu.make_async_copy(
                    table_hbm.at[idx_v.at[b]], buf_v.at[b], sems.at[b]
                ).wait()

                @plsc.parallel_loop(0, CH, unroll=8)
                def _rows(i):
                    for h in range(D // LANES):
                        v = buf_v[b, i, pl.ds(h * LANES, LANES)]
                        buf_v[b, i, pl.ds(h * LANES, LANES)] = _tanh16(v)
                pltpu.sync_copy(
                    buf_v.at[b], out_hbm.at[pl.ds(base + (ch + b) * CH, CH)]
                )
                nxt = ch + b + NBUF

                @pl.when(nxt < n_ch)
                def _():
                    pltpu.async_copy(
                        table_hbm.at[idx_v.at[nxt]], buf_v.at[b], sems.at[b]
                    )

            return carry

        lax.fori_loop(0, n_ch // NBUF, lambda g, c: chunk_body(g * NBUF, c), 0)

    return k


def kernel(x, table):
    B, S = x.shape
    V, D = table.shape
    N = B * S
    n_per_w = N // NW
    n_ch = n_per_w // CH
    xw = x.reshape(NW, n_ch, CH).astype(jnp.int32)
    out = _build(N, D, n_per_w, n_ch)(xw, table)
    return out.reshape(B, S, D)


# speedup vs baseline: 1.5722x; 1.5552x over previous
import functools

import jax
import jax.numpy as jnp
from jax import lax
from jax.experimental import pallas as pl
from jax.experimental.pallas import tpu as pltpu
from jax.experimental.pallas import tpu_sc as plsc

NC = 2
NS = 16
NW = NC * NS
LANES = 16
NBUF = 4
PREF = 2


def _tanh16(v):
    c = jnp.maximum(v, -10.0)
    e = jnp.exp(-2.0 * c)
    return (1.0 - e) / (1.0 + e)


@functools.lru_cache(maxsize=None)
def _build(B, S, V, D):
    Q = B // NW
    mesh = plsc.VectorSubcoreMesh(core_axis_name="c", subcore_axis_name="s")

    @functools.partial(
        pl.kernel,
        out_type=jax.ShapeDtypeStruct((B, S, D), jnp.float32),
        mesh=mesh,
        scratch_types=[
            pltpu.VMEM((Q, S), jnp.int32),
            pltpu.VMEM((NBUF, S, D), jnp.float32),
            pltpu.SemaphoreType.DMA((NBUF,)),
            pltpu.SemaphoreType.DMA((NBUF,)),
        ],
        compiler_params=pltpu.CompilerParams(use_tc_tiling_on_sc=False),
    )
    def k(x_hbm, table_hbm, out_hbm, idx_v, buf_v, gsem, ssem):
        wid = lax.axis_index("s") * NC + lax.axis_index("c")
        q0 = wid * Q
        pltpu.sync_copy(x_hbm.at[pl.ds(q0, Q)], idx_v)
        for b in range(PREF):
            pltpu.async_copy(table_hbm.at[idx_v.at[b]], buf_v.at[b], gsem.at[b])

        def q_body(g, carry):
            for b in range(NBUF):
                q = g * NBUF + b
                b2 = (b + PREF) % NBUF
                nxt = q + PREF

                @pl.when(nxt < Q)
                def _():
                    @pl.when(nxt >= NBUF)
                    def _():
                        pltpu.make_async_copy(
                            buf_v.at[b2],
                            out_hbm.at[q0 + nxt - NBUF],
                            ssem.at[b2],
                        ).wait()

                    pltpu.async_copy(
                        table_hbm.at[idx_v.at[nxt]], buf_v.at[b2], gsem.at[b2]
                    )

                pltpu.make_async_copy(
                    table_hbm.at[idx_v.at[q]], buf_v.at[b], gsem.at[b]
                ).wait()

                @plsc.parallel_loop(0, S, unroll=8)
                def _rows(i):
                    for h in range(D // LANES):
                        v = buf_v[b, i, pl.ds(h * LANES, LANES)]
                        buf_v[b, i, pl.ds(h * LANES, LANES)] = _tanh16(v)

                pltpu.async_copy(buf_v.at[b], out_hbm.at[q0 + q], ssem.at[b])
            return carry

        lax.fori_loop(0, Q // NBUF, q_body, 0)
        for b in range(NBUF):
            pltpu.make_async_copy(
                buf_v.at[b], out_hbm.at[q0 + Q - NBUF + b], ssem.at[b]
            ).wait()

    return k


def kernel(x, table):
    B, S = x.shape
    V, D = table.shape
    return _build(B, S, V, D)(x.astype(jnp.int32), table)


# speedup vs baseline: 1.8650x; 1.1862x over previous
import functools

import jax
import jax.numpy as jnp
from jax import lax
from jax.experimental import pallas as pl
from jax.experimental.pallas import tpu as pltpu
from jax.experimental.pallas import tpu_sc as plsc

NC = 2
NS = 16
NW = NC * NS
LANES = 16
BLK = 128
NBUF = 4
PREF = 2


def _tanh16(v):
    e = jnp.exp(jnp.minimum(-2.0 * v, 20.0))
    return (1.0 - e) / (1.0 + e)


@functools.lru_cache(maxsize=None)
def _build(B, S, V, D):
    QW = NBUF * BLK
    DT = D // 8
    mesh = plsc.VectorSubcoreMesh(core_axis_name="c", subcore_axis_name="s")

    @functools.partial(
        pl.kernel,
        out_type=jax.ShapeDtypeStruct((S, DT, B // BLK, 8, BLK), jnp.float32),
        mesh=mesh,
        scratch_types=[
            pltpu.VMEM((QW, S), jnp.int32),
            pltpu.VMEM((S, QW), jnp.int32),
            pltpu.VMEM((NBUF, BLK, D), jnp.float32),
            pltpu.VMEM((NBUF, DT, 8, BLK), jnp.float32),
            pltpu.SemaphoreType.DMA((NBUF,)),
            pltpu.SemaphoreType.DMA((NBUF,)),
        ],
        compiler_params=pltpu.CompilerParams(
            use_tc_tiling_on_sc=False, needs_layout_passes=False
        ),
    )
    def k(x_hbm, table_hbm, y_hbm, xbuf, idx_v, gbuf, obuf, gsem, ssem):
        wid = lax.axis_index("s") * NC + lax.axis_index("c")
        q0 = wid * QW
        lane = lax.iota(jnp.int32, LANES)

        pltpu.sync_copy(x_hbm.at[pl.ds(q0, QW)], xbuf)

        @plsc.parallel_loop(0, S * (QW // LANES), unroll=4)
        def _tr(i):
            s = i // (QW // LANES)
            r16 = i % (QW // LANES)
            rows = r16 * LANES + lane
            cols = jnp.full((LANES,), s, jnp.int32)
            idx_v[s, pl.ds(r16 * LANES, LANES)] = plsc.load_gather(
                xbuf, [rows, cols]
            )

        cg_rows = [jnp.full((LANES,), cg * LANES, jnp.int32) + lane
                   for cg in range(BLK // LANES)]

        def gather(s, u):
            pltpu.async_copy(
                table_hbm.at[idx_v.at[s, pl.ds(u * BLK, BLK)]],
                gbuf.at[u],
                gsem.at[u],
            )

        for u in range(PREF):
            gather(0, u)

        def s_body(s, carry):
            for u in range(NBUF):
                tc = wid * NBUF + u
                nu = (u + PREF) % NBUF
                ns = s + (u + PREF) // NBUF

                @pl.when(ns < S)
                def _():
                    gather(ns, nu)

                pltpu.make_async_copy(
                    table_hbm.at[idx_v.at[s, pl.ds(u * BLK, BLK)]],
                    gbuf.at[u],
                    gsem.at[u],
                ).wait()

                @pl.when(s > 0)
                def _():
                    for tr in range(DT):
                        pltpu.make_async_copy(
                            obuf.at[u, tr],
                            y_hbm.at[s - 1, tr, tc],
                            ssem.at[u],
                        ).wait()

                @plsc.parallel_loop(0, D, unroll=4)
                def _rows(d):
                    tr = d // 8
                    r = d % 8
                    cols = jnp.full((LANES,), d, jnp.int32)
                    for cg in range(BLK // LANES):
                        v = plsc.load_gather(gbuf.at[u], [cg_rows[cg], cols])
                        obuf[u, tr, r, pl.ds(cg * LANES, LANES)] = _tanh16(v)

                for tr in range(DT):
                    pltpu.async_copy(
                        obuf.at[u, tr], y_hbm.at[s, tr, tc], ssem.at[u]
                    )
            return carry

        lax.fori_loop(0, S, s_body, 0)
        for u in range(NBUF):
            for tr in range(DT):
                pltpu.make_async_copy(
                    obuf.at[u, tr],
                    y_hbm.at[S - 1, tr, wid * NBUF + u],
                    ssem.at[u],
                ).wait()

    return k


def kernel(x, table):
    B, S = x.shape
    V, D = table.shape
    y = _build(B, S, V, D)(x.astype(jnp.int32), table)
    return y.transpose(2, 4, 0, 1, 3).reshape(B, S, D)


# speedup vs baseline: 3.0701x; 1.6461x over previous
import functools

import jax
import jax.numpy as jnp
from jax import lax
from jax.experimental import pallas as pl
from jax.experimental.pallas import tpu as pltpu
from jax.experimental.pallas import tpu_sc as plsc


_UNT_BV = 2048


def _untile_body(tt_ref, out_ref):
    d, bv = tt_ref.shape
    m = (bv * d) // 128
    out_ref[...] = jnp.concatenate(
        [tt_ref[:, q * m:(q + 1) * m].T for q in range(128 // d)], axis=1
    )


@functools.lru_cache(maxsize=None)
def _build_untile(V, D):
    bv = _UNT_BV
    grid = (V + bv - 1) // bv
    m = (bv * D) // 128
    return pl.pallas_call(
        _untile_body,
        grid=(grid,),
        in_specs=[pl.BlockSpec((D, bv), lambda i: (0, i))],
        out_specs=pl.BlockSpec((m, 128), lambda i: (i, 0)),
        out_shape=jax.ShapeDtypeStruct((grid * m, 128), jnp.float32),
    )

NC = 2
NS = 16
NW = NC * NS
LANES = 16
BLK = 128
NBUF = 4
PREF = 2


def _tanh16(v):
    e = jnp.exp(jnp.minimum(-2.0 * v, 20.0))
    return (1.0 - e) / (1.0 + e)


@functools.lru_cache(maxsize=None)
def _build(B, S, V, D):
    QW = NBUF * BLK
    DT = D // 8
    mesh = plsc.VectorSubcoreMesh(core_axis_name="c", subcore_axis_name="s")

    @functools.partial(
        pl.kernel,
        out_type=jax.ShapeDtypeStruct((S, DT, B // BLK, 8, BLK), jnp.float32),
        mesh=mesh,
        scratch_types=[
            pltpu.VMEM((QW, S), jnp.int32),
            pltpu.VMEM((S, QW), jnp.int32),
            pltpu.VMEM((NBUF, BLK, D), jnp.float32),
            pltpu.VMEM((NBUF, D, BLK + 1), jnp.float32),
            pltpu.SemaphoreType.DMA((NBUF,)),
            pltpu.SemaphoreType.DMA((NBUF,)),
        ],
        compiler_params=pltpu.CompilerParams(
            use_tc_tiling_on_sc=False, needs_layout_passes=False
        ),
    )
    def k(x_hbm, table_hbm, y_hbm, xbuf, idx_v, gbuf, obuf, gsem, ssem):
        wid = lax.axis_index("s") * NC + lax.axis_index("c")
        q0 = wid * QW
        lane = lax.iota(jnp.int32, LANES)

        pltpu.sync_copy(x_hbm.at[pl.ds(q0, QW)], xbuf)

        @plsc.parallel_loop(0, S * (QW // LANES), unroll=4)
        def _tr(i):
            s = i // (QW // LANES)
            r16 = i % (QW // LANES)
            rows = r16 * LANES + lane
            cols = jnp.full((LANES,), s, jnp.int32)
            v = plsc.load_gather(xbuf, [rows, cols])
            g = (v & -2048) + ((v & 511) * 4) + ((v >> 9) & 3)
            idx_v[s, pl.ds(r16 * LANES, LANES)] = g

        dg_rows = [jnp.full((LANES,), dg * LANES, jnp.int32) + lane
                   for dg in range(D // LANES)]

        def gather(s, u):
            pltpu.async_copy(
                table_hbm.at[idx_v.at[s, pl.ds(u * BLK, BLK)]],
                gbuf.at[u],
                gsem.at[u],
            )


        for u in range(PREF):
            gather(0, u)

        def s_body(s, carry):
            for u in range(NBUF):
                tc = wid * NBUF + u
                nu = (u + PREF) % NBUF
                ns = s + (u + PREF) // NBUF

                @pl.when(ns < S)
                def _():
                    gather(ns, nu)

                pltpu.make_async_copy(
                    table_hbm.at[idx_v.at[s, pl.ds(u * BLK, BLK)]],
                    gbuf.at[u],
                    gsem.at[u],
                ).wait()

                @pl.when(s > 0)
                def _():
                    for tr in range(DT):
                        pltpu.make_async_copy(
                            obuf.at[u, pl.ds(tr * 8, 8), pl.ds(0, BLK)],
                            y_hbm.at[s - 1, tr, tc],
                            ssem.at[u],
                        ).wait()

                @plsc.parallel_loop(0, BLK, unroll=4)
                def _cols(c):
                    cvec = jnp.full((LANES,), c, jnp.int32)
                    for dg in range(D // LANES):
                        v = gbuf[u, c, pl.ds(dg * LANES, LANES)]
                        plsc.store_scatter(
                            obuf.at[u], [dg_rows[dg], cvec], _tanh16(v)
                        )

                for tr in range(DT):
                    pltpu.async_copy(
                        obuf.at[u, pl.ds(tr * 8, 8), pl.ds(0, BLK)],
                        y_hbm.at[s, tr, tc],
                        ssem.at[u],
                    )
            return carry

        lax.fori_loop(0, S, s_body, 0)
        for u in range(NBUF):
            for tr in range(DT):
                pltpu.make_async_copy(
                    obuf.at[u, pl.ds(tr * 8, 8), pl.ds(0, BLK)],
                    y_hbm.at[S - 1, tr, wid * NBUF + u],
                    ssem.at[u],
                ).wait()

    return k


def kernel(x, table):
    B, S = x.shape
    V, D = table.shape
    lin = _build_untile(V, D)(table.T).reshape(-1, D)
    y = _build(B, S, V, D)(x.astype(jnp.int32), lin)
    return y.transpose(2, 4, 0, 1, 3).reshape(B, S, D)


# speedup vs baseline: 4.3053x; 1.4023x over previous
import functools

import jax
import jax.numpy as jnp
from jax import lax
from jax.experimental import pallas as pl
from jax.experimental.pallas import tpu as pltpu
from jax.experimental.pallas import tpu_sc as plsc


_UNT_BV = 16384


def _untile_body(tt_ref, out_ref):
    d, bv = tt_ref.shape
    m = (bv * d) // 128
    out_ref[...] = jnp.concatenate(
        [tt_ref[:, q * m:(q + 1) * m].T for q in range(128 // d)], axis=1
    )


@functools.lru_cache(maxsize=None)
def _build_untile(V, D):
    bv = _UNT_BV
    grid = (V + bv - 1) // bv
    m = (bv * D) // 128
    return pl.pallas_call(
        _untile_body,
        grid=(grid,),
        in_specs=[pl.BlockSpec((D, bv), lambda i: (0, i))],
        out_specs=pl.BlockSpec((m, 128), lambda i: (i, 0)),
        out_shape=jax.ShapeDtypeStruct((grid * m, 128), jnp.float32),
    )

NC = 2
NS = 16
NW = NC * NS
LANES = 16
BLK = 128
NBUF = 4
PREF = 2


def _tanh16(v):
    e = jnp.exp(jnp.minimum(-2.0 * v, 20.0))
    return (1.0 - e) / (1.0 + e)


@functools.lru_cache(maxsize=None)
def _build(B, S, V, D):
    QW = NBUF * BLK
    DT = D // 8
    mb = (_UNT_BV * D) // 128
    qn = 128 // D
    msh = mb.bit_length() - 1
    mesh = plsc.VectorSubcoreMesh(core_axis_name="c", subcore_axis_name="s")

    @functools.partial(
        pl.kernel,
        out_type=jax.ShapeDtypeStruct((S, DT, B // BLK, 8, BLK), jnp.float32),
        mesh=mesh,
        scratch_types=[
            pltpu.VMEM((QW, S), jnp.int32),
            pltpu.VMEM((S, QW), jnp.int32),
            pltpu.VMEM((NBUF, BLK, D), jnp.float32),
            pltpu.VMEM((NBUF, D, BLK + 1), jnp.float32),
            pltpu.SemaphoreType.DMA((NBUF,)),
            pltpu.SemaphoreType.DMA((NBUF,)),
        ],
        compiler_params=pltpu.CompilerParams(
            use_tc_tiling_on_sc=False, needs_layout_passes=False
        ),
    )
    def k(x_hbm, table_hbm, y_hbm, xbuf, idx_v, gbuf, obuf, gsem, ssem):
        wid = lax.axis_index("s") * NC + lax.axis_index("c")
        q0 = wid * QW
        lane = lax.iota(jnp.int32, LANES)

        pltpu.sync_copy(x_hbm.at[pl.ds(q0, QW)], xbuf)

        @plsc.parallel_loop(0, S * (QW // LANES), unroll=4)
        def _tr(i):
            s = i // (QW // LANES)
            r16 = i % (QW // LANES)
            rows = r16 * LANES + lane
            cols = jnp.full((LANES,), s, jnp.int32)
            v = plsc.load_gather(xbuf, [rows, cols])
            g = (v & -_UNT_BV) + ((v & (mb - 1)) * qn) + ((v >> msh) & (qn - 1))
            idx_v[s, pl.ds(r16 * LANES, LANES)] = g

        dg_rows = [jnp.full((LANES,), dg * LANES, jnp.int32) + lane
                   for dg in range(D // LANES)]

        def gather(s, u):
            pltpu.async_copy(
                table_hbm.at[idx_v.at[s, pl.ds(u * BLK, BLK)]],
                gbuf.at[u],
                gsem.at[u],
            )


        for u in range(PREF):
            gather(0, u)

        def s_body(s, carry):
            for u in range(NBUF):
                tc = wid * NBUF + u
                nu = (u + PREF) % NBUF
                ns = s + (u + PREF) // NBUF

                @pl.when(ns < S)
                def _():
                    gather(ns, nu)

                pltpu.make_async_copy(
                    table_hbm.at[idx_v.at[s, pl.ds(u * BLK, BLK)]],
                    gbuf.at[u],
                    gsem.at[u],
                ).wait()

                @pl.when(s > 0)
                def _():
                    for tr in range(DT):
                        pltpu.make_async_copy(
                            obuf.at[u, pl.ds(tr * 8, 8), pl.ds(0, BLK)],
                            y_hbm.at[s - 1, tr, tc],
                            ssem.at[u],
                        ).wait()

                @plsc.parallel_loop(0, BLK, unroll=4)
                def _cols(c):
                    cvec = jnp.full((LANES,), c, jnp.int32)
                    for dg in range(D // LANES):
                        v = gbuf[u, c, pl.ds(dg * LANES, LANES)]
                        plsc.store_scatter(
                            obuf.at[u], [dg_rows[dg], cvec], _tanh16(v)
                        )

                for tr in range(DT):
                    pltpu.async_copy(
                        obuf.at[u, pl.ds(tr * 8, 8), pl.ds(0, BLK)],
                        y_hbm.at[s, tr, tc],
                        ssem.at[u],
                    )
            return carry

        lax.fori_loop(0, S, s_body, 0)
        for u in range(NBUF):
            for tr in range(DT):
                pltpu.make_async_copy(
                    obuf.at[u, pl.ds(tr * 8, 8), pl.ds(0, BLK)],
                    y_hbm.at[S - 1, tr, wid * NBUF + u],
                    ssem.at[u],
                ).wait()

    return k


def kernel(x, table):
    B, S = x.shape
    V, D = table.shape
    lin = _build_untile(V, D)(table.T).reshape(-1, D)
    y = _build(B, S, V, D)(x.astype(jnp.int32), lin)
    return y.transpose(2, 4, 0, 1, 3).reshape(B, S, D)
